# baseline (speedup 1.0000x reference)
import functools

import jax
import jax.numpy as jnp
from jax import lax
from jax.experimental import pallas as pl
from jax.experimental.pallas import tpu as pltpu
from jax.experimental.pallas import tpu_sc as plsc

NUM_ENT = 1000000
NUM_REL = 512
D = 64
B = 16384

_INFO = plsc.get_sparse_core_info()
NC = _INFO.num_cores
NS = _INFO.num_subcores
L = _INFO.num_lanes
NW = NC * NS
BPW = B // NW
CH = 128
NCH = BPW // CH


@functools.partial(
    pl.kernel,
    out_type=[
        jax.ShapeDtypeStruct((B,), jnp.float32),
        jax.ShapeDtypeStruct((B, D), jnp.float32),
    ],
    mesh=plsc.VectorSubcoreMesh(core_axis_name="c", subcore_axis_name="s"),
    compiler_params=pltpu.CompilerParams(needs_layout_passes=False),
    scratch_types=[
        pltpu.VMEM((BPW,), jnp.int32),
        pltpu.VMEM((BPW,), jnp.int32),
        pltpu.VMEM((BPW,), jnp.int32),
        pltpu.VMEM((BPW,), jnp.int32),
        pltpu.VMEM((NCH, CH), jnp.int32),
        pltpu.VMEM((NCH, CH), jnp.int32),
        pltpu.VMEM((BPW,), jnp.int32),
        pltpu.VMEM((BPW,), jnp.int32),
        pltpu.VMEM((CH, 2 * D), jnp.float32),
        pltpu.VMEM((CH, 2 * D), jnp.float32),
        pltpu.VMEM((NUM_REL * D,), jnp.float32),
        pltpu.VMEM((CH, D), jnp.float32),
        pltpu.VMEM((BPW,), jnp.float32),
        pltpu.SemaphoreType.DMA,
    ],
)
def _sc_kernel(heads_hbm, rels_hbm, tails_hbm, mask_hbm, ent2_hbm, rel_hbm,
               scores_hbm, new_hbm,
               h_v, r_v, t_v, mask_v, obs_pr, new_pr, obs_hf, new_hf,
               obs_pair, new_pair, rel_flat, new_stage, scores_v, sem):
    wid = lax.axis_index("s") * NC + lax.axis_index("c")
    base = wid * BPW

    pltpu.sync_copy(heads_hbm.at[pl.ds(base, BPW)], h_v)
    pltpu.sync_copy(rels_hbm.at[pl.ds(base, BPW)], r_v)
    pltpu.sync_copy(tails_hbm.at[pl.ds(base, BPW)], t_v)
    pltpu.sync_copy(mask_hbm.at[pl.ds(base, BPW)], mask_v)
    rel_cp = pltpu.async_copy(rel_hbm, rel_flat, sem)

    for i in range(BPW // L):
        sl = pl.ds(i * L, L)
        m = mask_v[sl]
        h = h_v[sl]
        t = t_v[sl]
        is0 = m == 0
        obs_ids = jnp.where(is0, t, h)
        new_ids = jnp.where(is0, h, t)
        r, c = (i * L) // CH, (i * L) % CH
        obs_pr[r, pl.ds(c, L)] = obs_ids >> 1
        new_pr[r, pl.ds(c, L)] = new_ids >> 1
        obs_hf[sl] = (obs_ids & 1) * D
        new_hf[sl] = (new_ids & 1) * D

    rel_cp.wait()

    lane = lax.iota(jnp.int32, L)
    for c in range(NCH):
        cp_o = pltpu.async_copy(ent2_hbm.at[obs_pr.at[c]], obs_pair, sem)
        cp_n = pltpu.async_copy(ent2_hbm.at[new_pr.at[c]], new_pair, sem)
        cp_o.wait()
        cp_n.wait()

        def gbody(gg, carry, c=c):
            acc16 = jnp.zeros((L,), jnp.float32)
            gsl = pl.ds(c * CH + gg * L, L)
            rid_vec = r_v[gsl]
            ohf_vec = obs_hf[gsl]
            nhf_vec = new_hf[gsl]
            for j in range(L):
                rr = gg * L + j
                rid = rid_vec[j]
                roff = rid * D
                oh = ohf_vec[j]
                nh = nhf_vec[j]
                acc = None
                for k in range(D // L):
                    o = obs_pair[rr, pl.ds(oh + k * L, L)]
                    n = new_pair[rr, pl.ds(nh + k * L, L)]
                    rl = rel_flat[pl.ds(roff + k * L, L)]
                    new_stage[rr, pl.ds(k * L, L)] = n
                    p = o * n * rl
                    acc = p if acc is None else acc + p
                acc16 = jnp.where(lane == j, jnp.sum(acc), acc16)
            scores_v[pl.ds(c * CH + gg * L, L)] = acc16
            return carry

        lax.fori_loop(0, CH // L, gbody, 0)
        pltpu.sync_copy(new_stage, new_hbm.at[pl.ds(base + c * CH, CH)])

    pltpu.sync_copy(scores_v, scores_hbm.at[pl.ds(base, BPW)])


def kernel(triples, mask, ent_emb, rel_emb):
    tt = triples.T
    ent2 = ent_emb.reshape(NUM_ENT // 2, 2 * D)
    scores, new_embs = _sc_kernel(tt[0], tt[1], tt[2], mask,
                                  ent2, rel_emb.reshape(-1))
    return scores, new_embs
